# baseline (speedup 1.0000x reference)
import functools

import jax
import jax.numpy as jnp
from jax import lax
from jax.experimental import pallas as pl
from jax.experimental.pallas import tpu as pltpu
from jax.experimental.pallas import tpu_sc as plsc

NUM_EMB = 8192
DIM = 256
NUM_TOK = 8192
BLK = 512
GRID = NUM_TOK // BLK
COMMIT = 0.25
DIVW = 0.1

NW = 32
ROWS_W = NUM_TOK // NW


def _vq_tc_body(z_ref, et_ref, z2_ref, e2_ref, idx_ref, loss_ref,
                perp_ref, accp_ref, accl_ref):
    i = pl.program_id(0)

    @pl.when(i == 0)
    def _init():
        accp_ref[...] = jnp.zeros_like(accp_ref)
        accl_ref[...] = jnp.zeros_like(accl_ref)

    z = z_ref[...]
    mm = lax.dot_general(z, et_ref[...], (((1,), (0,)), ((), ())),
                         preferred_element_type=jnp.float32)
    d = (z2_ref[...] + e2_ref[...]) - 2.0 * mm
    m = jnp.min(d, axis=1, keepdims=True)
    colf = lax.broadcasted_iota(jnp.int32, d.shape, 1).astype(jnp.float32)
    idxf = jnp.min(jnp.where(d == m, colf, jnp.float32(NUM_EMB)), axis=1,
                   keepdims=True)
    idx_ref[...] = lax.transpose(idxf.astype(jnp.int32), (1, 0))
    e = jnp.exp(m - d)
    s = jnp.sum(e, axis=1, keepdims=True)
    accp_ref[...] += jnp.sum(e / s, axis=0, keepdims=True)
    accl_ref[...] += jnp.sum(m, keepdims=True)

    @pl.when(i == GRID - 1)
    def _fini():
        ap = accp_ref[...] / NUM_TOK
        ld = jnp.sum(ap * jnp.log(ap + 1e-10), keepdims=True)
        lvq = (1.0 + COMMIT) * accl_ref[...] / (NUM_TOK * DIM)
        loss_ref[...] = lvq + DIVW * ld
        perp_ref[...] = jnp.exp(-ld)


_vq_tc = pl.pallas_call(
    _vq_tc_body,
    grid=(GRID,),
    in_specs=[
        pl.BlockSpec((BLK, DIM), lambda i: (i, 0)),
        pl.BlockSpec((DIM, NUM_EMB), lambda i: (0, 0)),
        pl.BlockSpec((BLK, 1), lambda i: (i, 0)),
        pl.BlockSpec((1, NUM_EMB), lambda i: (0, 0)),
    ],
    out_specs=[
        pl.BlockSpec((1, BLK), lambda i: (0, i)),
        pl.BlockSpec((1, 1), lambda i: (0, 0)),
        pl.BlockSpec((1, 1), lambda i: (0, 0)),
    ],
    out_shape=[
        jax.ShapeDtypeStruct((1, NUM_TOK), jnp.int32),
        jax.ShapeDtypeStruct((1, 1), jnp.float32),
        jax.ShapeDtypeStruct((1, 1), jnp.float32),
    ],
    scratch_shapes=[
        pltpu.VMEM((1, NUM_EMB), jnp.float32),
        pltpu.VMEM((1, 1), jnp.float32),
    ],
)


def _sc_body(table_hbm, idx_hbm, out_hbm, idx_v, rows_v, sem):
    wid = lax.axis_index("s") * 2 + lax.axis_index("c")
    base = wid * ROWS_W
    pltpu.sync_copy(idx_hbm.at[pl.ds(base, ROWS_W)], idx_v)
    pltpu.async_copy(table_hbm.at[idx_v], rows_v, sem).wait()
    pltpu.sync_copy(rows_v, out_hbm.at[pl.ds(base, ROWS_W)])


@functools.lru_cache(maxsize=1)
def _sc_gather_st():
    return pl.kernel(
        _sc_body,
        mesh=plsc.VectorSubcoreMesh(core_axis_name="c", subcore_axis_name="s"),
        out_type=jax.ShapeDtypeStruct((NUM_TOK, DIM), jnp.float32),
        scratch_types=[
            pltpu.VMEM((ROWS_W,), jnp.int32),
            pltpu.VMEM((ROWS_W, DIM), jnp.float32),
            pltpu.SemaphoreType.DMA,
        ],
    )


def kernel(inputs, embedding_weight):
    flat = inputs.reshape(-1, DIM)
    z2 = jnp.sum(flat ** 2, axis=1, keepdims=True)
    e2 = jnp.sum(embedding_weight ** 2, axis=1)[None, :]
    idx, loss, perp = _vq_tc(flat, embedding_weight.T, z2, e2)
    idx_flat = idx.reshape(-1)
    qst = _sc_gather_st()(embedding_weight, idx_flat)
    return (qst.reshape(inputs.shape), loss[0, 0],
            idx.reshape(inputs.shape[0], inputs.shape[1]), perp[0, 0])
